# baseline (speedup 1.0000x reference)
import functools

import jax
import jax.numpy as jnp
from jax import lax
from jax.experimental import pallas as pl
from jax.experimental.pallas import tpu as pltpu
from jax.experimental.pallas import tpu_sc as plsc

HIDDEN = 768
NUM_EXPERTS = 64
TOP_K = 2
BT = 4096
T = 32768
NCHUNK = 2
TCH = T // NCHUNK
CB = TCH // BT

NC, NS, L = 2, 16, 16
NW = NC * NS
TPW = TCH // NW
GROUPS = TPW // L
GI = 4


def _logits_first(x_ref, wt_ref, b_ref, logits_ref, logits_t_ref):
    logits = jax.lax.dot_general(
        x_ref[...], wt_ref[...], (((1,), (0,)), ((), ())),
        preferred_element_type=jnp.float32,
    ) + b_ref[...][None, :]
    logits_ref[...] = logits
    logits_t_ref[...] = logits.T


def _logits_next(x_ref, wt_ref, b_ref, buf_ref, logits_ref, logits_t_ref):
    del buf_ref
    _logits_first(x_ref, wt_ref, b_ref, logits_ref, logits_t_ref)


def _tc_chunk(c, buf, x, wt, b):
    in_specs = [
        pl.BlockSpec((BT, HIDDEN), lambda i, c=c: (c * CB + i, 0)),
        pl.BlockSpec((HIDDEN, NUM_EXPERTS), lambda i: (0, 0)),
        pl.BlockSpec((NUM_EXPERTS,), lambda i: (0,)),
    ]
    out_specs = [
        pl.BlockSpec((BT, NUM_EXPERTS), lambda i, c=c: (c * CB + i, 0)),
        pl.BlockSpec((NUM_EXPERTS, BT), lambda i: (0, i)),
    ]
    out_shape = [
        jax.ShapeDtypeStruct((T, NUM_EXPERTS), jnp.float32),
        jax.ShapeDtypeStruct((NUM_EXPERTS, TCH), jnp.float32),
    ]
    if buf is None:
        return pl.pallas_call(
            _logits_first, grid=(CB,), in_specs=in_specs,
            out_specs=out_specs, out_shape=out_shape,
        )(x, wt, b)
    return pl.pallas_call(
        _logits_next, grid=(CB,),
        in_specs=in_specs + [pl.BlockSpec(memory_space=pl.ANY)],
        out_specs=out_specs, out_shape=out_shape,
        input_output_aliases={3: 0},
    )(x, wt, b, buf)


def _route_body(logits_t_hbm, w_hbm, e_hbm, chunk_v, w_v, e_v):
    cid = lax.axis_index("c")
    sid = lax.axis_index("s")
    wid = sid * NC + cid
    base = wid * TPW
    pltpu.sync_copy(logits_t_hbm.at[:, pl.ds(base, TPW)], chunk_v)

    lane = lax.iota(jnp.int32, L)

    def group(g0, _):
        neg = jnp.full((L,), -jnp.inf, jnp.float32)
        big = jnp.full((L,), NUM_EXPERTS, jnp.int32)

        st = [[neg, neg, big, big] for _ in range(GI)]
        for e in range(NUM_EXPERTS):
            ev = jnp.full((L,), e, jnp.int32)
            for j in range(GI):
                m1, m2, a1, a2 = st[j]
                v = chunk_v[e, pl.ds((g0 + j) * L, L)]
                gt1 = v > m1
                gt2 = v > m2
                a2 = jnp.where(gt1, a1, jnp.where(gt2, ev, a2))
                m2 = jnp.where(gt1, m1, jnp.where(gt2, v, m2))
                a1 = jnp.where(gt1, ev, a1)
                m1 = jnp.where(gt1, v, m1)
                st[j] = [m1, m2, a1, a2]
        for j in range(GI):
            m1, m2, a1, a2 = st[j]
            w1 = 1.0 / (1.0 + jnp.exp(m2 - m1))
            w2 = 1.0 - w1
            pos = ((g0 + j) * L + lane) * 2
            plsc.store_scatter(w_v, [pos], w1)
            plsc.store_scatter(w_v, [pos + 1], w2)
            plsc.store_scatter(e_v, [pos], a1)
            plsc.store_scatter(e_v, [pos + 1], a2)
        return 0

    lax.fori_loop(0, GROUPS // GI, lambda i, c: group(i * GI, c), 0)
    pltpu.sync_copy(w_v, w_hbm.at[pl.ds(base * 2, 2 * TPW)])
    pltpu.sync_copy(e_v, e_hbm.at[pl.ds(base * 2, 2 * TPW)])


_route = pl.kernel(
    _route_body,
    out_type=[
        jax.ShapeDtypeStruct((2 * TCH,), jnp.float32),
        jax.ShapeDtypeStruct((2 * TCH,), jnp.int32),
    ],
    mesh=plsc.VectorSubcoreMesh(
        core_axis_name="c", subcore_axis_name="s",
        num_cores=NC, num_subcores=NS),
    scratch_types=[
        pltpu.VMEM((NUM_EXPERTS, TPW), jnp.float32),
        pltpu.VMEM((2 * TPW,), jnp.float32),
        pltpu.VMEM((2 * TPW,), jnp.int32),
    ],
    compiler_params=pltpu.CompilerParams(needs_layout_passes=False),
)


@jax.jit
def kernel(hidden_states, gate_w, gate_b):
    b, s, h = hidden_states.shape
    t = b * s
    x = hidden_states.reshape(t, h)
    wt = gate_w.T

    logits = None
    wfs, efs = [], []
    for c in range(NCHUNK):
        logits, logits_t_c = _tc_chunk(c, logits, x, wt, gate_b)
        wf_c, ef_c = _route(logits_t_c)
        wfs.append(wf_c)
        efs.append(ef_c)

    weights = jnp.concatenate(wfs).reshape(t, TOP_K)
    experts = jnp.concatenate(efs).reshape(t, TOP_K)
    aux_loss = jnp.array(0.0, dtype=jnp.float32)
    return (weights, experts, logits, aux_loss)


# speedup vs baseline: 1.0833x; 1.0833x over previous
import functools

import jax
import jax.numpy as jnp
from jax import lax
from jax.experimental import pallas as pl
from jax.experimental.pallas import tpu as pltpu
from jax.experimental.pallas import tpu_sc as plsc

HIDDEN = 768
NUM_EXPERTS = 64
TOP_K = 2
BT = 4096

T = 32768
NC, NS, L = 2, 16, 16
NW = NC * NS
TPW = T // NW
GROUPS = TPW // L
GI = 4


def _logits_block(x_ref, wt_ref, b_ref, logits_ref, logits_t_ref):
    logits = jax.lax.dot_general(
        x_ref[...], wt_ref[...], (((1,), (0,)), ((), ())),
        preferred_element_type=jnp.float32,
    ) + b_ref[...][None, :]
    logits_ref[...] = logits
    logits_t_ref[...] = logits.T


def _route_body(logits_t_hbm, w_hbm, e_hbm, chunk_v, w_v, e_v):
    cid = lax.axis_index("c")
    sid = lax.axis_index("s")
    wid = sid * NC + cid
    base = wid * TPW
    pltpu.sync_copy(logits_t_hbm.at[:, pl.ds(base, TPW)], chunk_v)

    lane = lax.iota(jnp.int32, L)

    def group(g0, _):
        neg = jnp.full((L,), -jnp.inf, jnp.float32)
        big = jnp.full((L,), NUM_EXPERTS, jnp.int32)

        st = [[neg, neg, big, big] for _ in range(GI)]
        for e in range(NUM_EXPERTS):
            ev = jnp.full((L,), e, jnp.int32)
            for j in range(GI):
                m1, m2, a1, a2 = st[j]
                v = chunk_v[e, pl.ds((g0 + j) * L, L)]
                gt1 = v > m1
                gt2 = v > m2
                a2 = jnp.where(gt1, a1, jnp.where(gt2, ev, a2))
                m2 = jnp.where(gt1, m1, jnp.where(gt2, v, m2))
                a1 = jnp.where(gt1, ev, a1)
                m1 = jnp.where(gt1, v, m1)
                st[j] = [m1, m2, a1, a2]
        for j in range(GI):
            m1, m2, a1, a2 = st[j]
            w1 = 1.0 / (1.0 + jnp.exp(m2 - m1))
            w2 = 1.0 - w1
            pos = ((g0 + j) * L + lane) * 2
            plsc.store_scatter(w_v, [pos], w1)
            plsc.store_scatter(w_v, [pos + 1], w2)
            plsc.store_scatter(e_v, [pos], a1)
            plsc.store_scatter(e_v, [pos + 1], a2)
        return 0

    lax.fori_loop(0, GROUPS // GI, lambda i, c: group(i * GI, c), 0)
    pltpu.sync_copy(w_v, w_hbm.at[pl.ds(base * 2, 2 * TPW)])
    pltpu.sync_copy(e_v, e_hbm.at[pl.ds(base * 2, 2 * TPW)])


_route = pl.kernel(
    _route_body,
    out_type=[
        jax.ShapeDtypeStruct((2 * T,), jnp.float32),
        jax.ShapeDtypeStruct((2 * T,), jnp.int32),
    ],
    mesh=plsc.VectorSubcoreMesh(
        core_axis_name="c", subcore_axis_name="s",
        num_cores=NC, num_subcores=NS),
    scratch_types=[
        pltpu.VMEM((NUM_EXPERTS, TPW), jnp.float32),
        pltpu.VMEM((2 * TPW,), jnp.float32),
        pltpu.VMEM((2 * TPW,), jnp.int32),
    ],
    compiler_params=pltpu.CompilerParams(needs_layout_passes=False),
)


@jax.jit
def kernel(hidden_states, gate_w, gate_b):
    b, s, h = hidden_states.shape
    t = b * s
    x = hidden_states.reshape(t, h)
    wt = gate_w.T

    logits, logits_t = pl.pallas_call(
        _logits_block,
        grid=(t // BT,),
        in_specs=[
            pl.BlockSpec((BT, h), lambda i: (i, 0)),
            pl.BlockSpec((h, NUM_EXPERTS), lambda i: (0, 0)),
            pl.BlockSpec((NUM_EXPERTS,), lambda i: (0,)),
        ],
        out_specs=[
            pl.BlockSpec((BT, NUM_EXPERTS), lambda i: (i, 0)),
            pl.BlockSpec((NUM_EXPERTS, BT), lambda i: (0, i)),
        ],
        out_shape=[
            jax.ShapeDtypeStruct((t, NUM_EXPERTS), jnp.float32),
            jax.ShapeDtypeStruct((NUM_EXPERTS, t), jnp.float32),
        ],
    )(x, wt, gate_b)

    wf, ef = _route(logits_t)
    weights = wf.reshape(t, TOP_K)
    experts = ef.reshape(t, TOP_K)
    aux_loss = jnp.array(0.0, dtype=jnp.float32)
    return (weights, experts, logits, aux_loss)


# speedup vs baseline: 1.0881x; 1.0045x over previous
import functools

import jax
import jax.numpy as jnp
from jax import lax
from jax.experimental import pallas as pl
from jax.experimental.pallas import tpu as pltpu
from jax.experimental.pallas import tpu_sc as plsc

HIDDEN = 768
NUM_EXPERTS = 64
TOP_K = 2
BT = 4096

T = 32768
NC, NS, L = 2, 16, 16
NW = NC * NS
TPW = T // NW
GROUPS = TPW // L
GI = 4


def _logits_block(x_ref, wt_ref, b_ref, logits_ref, logits_t_ref):
    logits = jax.lax.dot_general(
        x_ref[...], wt_ref[...], (((1,), (0,)), ((), ())),
        preferred_element_type=jnp.float32,
    ) + b_ref[...][None, :]
    logits_ref[...] = logits
    logits_t_ref[...] = logits.T


def _route_body(logits_t_hbm, w_hbm, e_hbm, chunk_v, w_v, e_v):
    cid = lax.axis_index("c")
    sid = lax.axis_index("s")
    wid = sid * NC + cid
    base = wid * TPW
    pltpu.sync_copy(logits_t_hbm.at[:, pl.ds(base, TPW)], chunk_v)

    lane = lax.iota(jnp.int32, L)

    def group(g0, _):
        neg = jnp.full((L,), -jnp.inf, jnp.float32)
        big = jnp.full((L,), NUM_EXPERTS, jnp.int32)

        st0 = [neg, neg, big, big] * GI

        def eblock(eb, carry):
            st = [list(carry[4 * j:4 * j + 4]) for j in range(GI)]
            for k in range(4):
                e = eb * 4 + k
                ev = jnp.full((L,), 1, jnp.int32) * e
                for j in range(GI):
                    m1, m2, a1, a2 = st[j]
                    v = chunk_v[e, pl.ds((g0 + j) * L, L)]
                    gt1 = v > m1
                    gt2 = v > m2
                    a2 = jnp.where(gt1, a1, jnp.where(gt2, ev, a2))
                    m2 = jnp.where(gt1, m1, jnp.where(gt2, v, m2))
                    a1 = jnp.where(gt1, ev, a1)
                    m1 = jnp.where(gt1, v, m1)
                    st[j] = [m1, m2, a1, a2]
            return tuple(x for s in st for x in s)

        stf = lax.fori_loop(0, NUM_EXPERTS // 4, eblock, tuple(st0))
        for j in range(GI):
            m1, m2, a1, a2 = stf[4 * j:4 * j + 4]
            w1 = 1.0 / (1.0 + jnp.exp(m2 - m1))
            w2 = 1.0 - w1
            pos = ((g0 + j) * L + lane) * 2
            plsc.store_scatter(w_v, [pos], w1)
            plsc.store_scatter(w_v, [pos + 1], w2)
            plsc.store_scatter(e_v, [pos], a1)
            plsc.store_scatter(e_v, [pos + 1], a2)
        return 0

    lax.fori_loop(0, GROUPS // GI, lambda i, c: group(i * GI, c), 0)
    pltpu.sync_copy(w_v, w_hbm.at[pl.ds(base * 2, 2 * TPW)])
    pltpu.sync_copy(e_v, e_hbm.at[pl.ds(base * 2, 2 * TPW)])


_route = pl.kernel(
    _route_body,
    out_type=[
        jax.ShapeDtypeStruct((2 * T,), jnp.float32),
        jax.ShapeDtypeStruct((2 * T,), jnp.int32),
    ],
    mesh=plsc.VectorSubcoreMesh(
        core_axis_name="c", subcore_axis_name="s",
        num_cores=NC, num_subcores=NS),
    scratch_types=[
        pltpu.VMEM((NUM_EXPERTS, TPW), jnp.float32),
        pltpu.VMEM((2 * TPW,), jnp.float32),
        pltpu.VMEM((2 * TPW,), jnp.int32),
    ],
    compiler_params=pltpu.CompilerParams(needs_layout_passes=False),
)


@jax.jit
def kernel(hidden_states, gate_w, gate_b):
    b, s, h = hidden_states.shape
    t = b * s
    x = hidden_states.reshape(t, h)
    wt = gate_w.T

    logits, logits_t = pl.pallas_call(
        _logits_block,
        grid=(t // BT,),
        in_specs=[
            pl.BlockSpec((BT, h), lambda i: (i, 0)),
            pl.BlockSpec((h, NUM_EXPERTS), lambda i: (0, 0)),
            pl.BlockSpec((NUM_EXPERTS,), lambda i: (0,)),
        ],
        out_specs=[
            pl.BlockSpec((BT, NUM_EXPERTS), lambda i: (i, 0)),
            pl.BlockSpec((NUM_EXPERTS, BT), lambda i: (0, i)),
        ],
        out_shape=[
            jax.ShapeDtypeStruct((t, NUM_EXPERTS), jnp.float32),
            jax.ShapeDtypeStruct((NUM_EXPERTS, t), jnp.float32),
        ],
    )(x, wt, gate_b)

    wf, ef = _route(logits_t)
    weights = wf.reshape(t, TOP_K)
    experts = ef.reshape(t, TOP_K)
    aux_loss = jnp.array(0.0, dtype=jnp.float32)
    return (weights, experts, logits, aux_loss)
